# Initial kernel scaffold; baseline (speedup 1.0000x reference)
import functools

import jax
import jax.numpy as jnp
from jax import lax
from jax.experimental import pallas as pl
from jax.experimental.pallas import tpu as pltpu
from jax.experimental.pallas import tpu_sc as plsc

B, N, S, K = 8, 4096, 512, 32
M = B * S * K
BIGI = jnp.int32(2**30)


def _fps_body(xyzT_ref, far0_ref, idx_ref, nx_ref, ny_ref, nz_ref):
    x = xyzT_ref[0]
    y = xyzT_ref[1]
    z = xyzT_ref[2]
    iota = lax.broadcasted_iota(jnp.int32, (B, N), 1)

    def body(i, carry):
        far, dist = carry
        idx_ref[:, pl.ds(i, 1)] = far
        oh = iota == far
        cx = jnp.sum(jnp.where(oh, x, 0.0), axis=1, keepdims=True)
        cy = jnp.sum(jnp.where(oh, y, 0.0), axis=1, keepdims=True)
        cz = jnp.sum(jnp.where(oh, z, 0.0), axis=1, keepdims=True)
        nx_ref[:, pl.ds(i, 1)] = cx
        ny_ref[:, pl.ds(i, 1)] = cy
        nz_ref[:, pl.ds(i, 1)] = cz
        dx = x - cx
        dy = y - cy
        dz = z - cz
        dn = dx * dx + dy * dy + dz * dz
        dist = jnp.minimum(dist, dn)
        m = jnp.max(dist, axis=1, keepdims=True)
        far = jnp.min(jnp.where(dist == m, iota, jnp.int32(N)), axis=1,
                      keepdims=True)
        return far, dist

    far0 = far0_ref[...]
    dist0 = jnp.full((B, N), 1e10, dtype=jnp.float32)
    lax.fori_loop(0, S, body, (far0, dist0))


def _fps(xyzT, far0):
    return pl.pallas_call(
        _fps_body,
        out_shape=[
            jax.ShapeDtypeStruct((B, S), jnp.int32),
            jax.ShapeDtypeStruct((B, S), jnp.float32),
            jax.ShapeDtypeStruct((B, S), jnp.float32),
            jax.ShapeDtypeStruct((B, S), jnp.float32),
        ],
    )(xyzT, far0)


def _knn_body(xyzT_ref, qT_ref, idx_ref):
    b = pl.program_id(0)
    px = xyzT_ref[0]
    py = xyzT_ref[1]
    pz = xyzT_ref[2]
    qx = qT_ref[0]
    qy = qT_ref[1]
    qz = qT_ref[2]
    a2 = qx * qx + qy * qy + qz * qz
    b2 = px * px + py * py + pz * pz
    q = jnp.concatenate([qx, qy, qz], axis=1)
    p = jnp.concatenate([px, py, pz], axis=0)
    qp = jnp.dot(q, p, preferred_element_type=jnp.float32)
    d = a2 + b2 - 2.0 * qp
    iota = lax.broadcasted_iota(jnp.int32, (S, N), 1)

    def body(r, dcur):
        m = jnp.min(dcur, axis=1, keepdims=True)
        am = jnp.min(jnp.where(dcur == m, iota, BIGI), axis=1, keepdims=True)
        idx_ref[0, :, pl.ds(r, 1)] = am + b * N
        return jnp.where(iota == am, jnp.float32(3e38), dcur)

    lax.fori_loop(0, K, body, d)


def _knn(xyzT, qT):
    return pl.pallas_call(
        _knn_body,
        grid=(B,),
        in_specs=[
            pl.BlockSpec((3, 1, N), lambda b: (0, b, 0)),
            pl.BlockSpec((3, S, 1), lambda b: (0, 0, b)),
        ],
        out_specs=pl.BlockSpec((1, S, K), lambda b: (b, 0, 0)),
        out_shape=jax.ShapeDtypeStruct((B, S, K), jnp.int32),
    )(xyzT, qT)


def _pret_body(xyz_ref, pts_ref, wx_ref, wp_ref, t_ref):
    t_ref[0] = (
        jnp.dot(xyz_ref[0], wx_ref[...], preferred_element_type=jnp.float32)
        + jnp.dot(pts_ref[0], wp_ref[...], preferred_element_type=jnp.float32)
    )


def _pretransform(xyz, points, w1xT, w1pT):
    return pl.pallas_call(
        _pret_body,
        grid=(B,),
        in_specs=[
            pl.BlockSpec((1, N, 3), lambda b: (b, 0, 0)),
            pl.BlockSpec((1, N, 64), lambda b: (b, 0, 0)),
            pl.BlockSpec((3, 64), lambda b: (0, 0)),
            pl.BlockSpec((64, 64), lambda b: (0, 0)),
        ],
        out_specs=pl.BlockSpec((1, N, 64), lambda b: (b, 0, 0)),
        out_shape=jax.ShapeDtypeStruct((B, N, 64), jnp.float32),
    )(xyz, points, w1xT, w1pT)


def _cq_body(ns_ref, wx_ref, b1_ref, cq_ref):
    cq_ref[...] = (
        jnp.dot(ns_ref[...], wx_ref[...], preferred_element_type=jnp.float32)
        - b1_ref[...]
    )


def _cq(ns, w1xT, b1row):
    return pl.pallas_call(
        _cq_body,
        out_shape=jax.ShapeDtypeStruct((B * S, 64), jnp.float32),
    )(ns, w1xT, b1row)


NWORK = 32
ROWS_W = M // NWORK
CH = 128
NCH = ROWS_W // CH


def _gather_sc(table, gidx):
    mesh = plsc.VectorSubcoreMesh(core_axis_name="c", subcore_axis_name="s")

    @functools.partial(
        pl.kernel,
        out_type=jax.ShapeDtypeStruct((M, 64), jnp.float32),
        mesh=mesh,
        scratch_types=[
            pltpu.VMEM((CH,), jnp.int32),
            pltpu.VMEM((CH, 64), jnp.float32),
            pltpu.SemaphoreType.DMA,
        ],
    )
    def k(table_hbm, idx_hbm, out_hbm, idx_v, rows_v, sem):
        wid = lax.axis_index("s") * 2 + lax.axis_index("c")
        base = wid * ROWS_W

        @pl.loop(0, NCH)
        def _(i):
            off = base + i * CH
            pltpu.sync_copy(idx_hbm.at[pl.ds(off, CH)], idx_v)
            pltpu.async_copy(table_hbm.at[idx_v], rows_v, sem).wait()
            pltpu.sync_copy(rows_v, out_hbm.at[pl.ds(off, CH)])

    return k(table, gidx)


RB = 4096
NB = M // RB
QB = RB // K


def _stats1_body(tg_ref, cq_ref, st_ref):
    @pl.when(pl.program_id(0) == 0)
    def _():
        st_ref[...] = jnp.zeros_like(st_ref)

    z = tg_ref[...].reshape(QB, K, 64) - cq_ref[...][:, None, :]
    s = jnp.sum(z, axis=(0, 1))[None, :]
    q = jnp.sum(z * z, axis=(0, 1))[None, :]
    st_ref[0:1, :] += s
    st_ref[1:2, :] += q


def _stats1(tg, cq):
    return pl.pallas_call(
        _stats1_body,
        grid=(NB,),
        in_specs=[
            pl.BlockSpec((RB, 64), lambda i: (i, 0)),
            pl.BlockSpec((QB, 64), lambda i: (i, 0)),
        ],
        out_specs=pl.BlockSpec((8, 64), lambda i: (0, 0)),
        out_shape=jax.ShapeDtypeStruct((8, 64), jnp.float32),
    )(tg, cq)


def _bn_apply(z, st_ref, g_ref, be_ref):
    s = st_ref[0:1, :]
    q = st_ref[1:2, :]
    mean = s / M
    var = q / M - mean * mean
    xh = (z - mean) / jnp.sqrt(var + 1e-5)
    return jnp.maximum(xh * g_ref[...] + be_ref[...], 0.0)


def _layer_body(in_ref, st_ref, g_ref, be_ref, w_ref, b_ref, cq_ref,
                z_ref, st2_ref, *, first):
    @pl.when(pl.program_id(0) == 0)
    def _():
        st2_ref[...] = jnp.zeros_like(st2_ref)

    zin = in_ref[...]
    if first:
        zin = (zin.reshape(QB, K, 64) - cq_ref[...][:, None, :]).reshape(
            RB, 64)
    y = _bn_apply(zin, st_ref, g_ref, be_ref)
    z = jnp.dot(y, w_ref[...], preferred_element_type=jnp.float32) + b_ref[...]
    z_ref[...] = z
    st2_ref[0:1, :] += jnp.sum(z, axis=0, keepdims=True)
    st2_ref[1:2, :] += jnp.sum(z * z, axis=0, keepdims=True)


def _layer(zin, st, g, be, wT, brow, cq, cin, cout, first):
    return pl.pallas_call(
        functools.partial(_layer_body, first=first),
        grid=(NB,),
        in_specs=[
            pl.BlockSpec((RB, cin), lambda i: (i, 0)),
            pl.BlockSpec((8, cin), lambda i: (0, 0)),
            pl.BlockSpec((1, cin), lambda i: (0, 0)),
            pl.BlockSpec((1, cin), lambda i: (0, 0)),
            pl.BlockSpec((cin, cout), lambda i: (0, 0)),
            pl.BlockSpec((1, cout), lambda i: (0, 0)),
            pl.BlockSpec((QB, 64), lambda i: (i, 0)),
        ],
        out_specs=[
            pl.BlockSpec((RB, cout), lambda i: (i, 0)),
            pl.BlockSpec((8, cout), lambda i: (0, 0)),
        ],
        out_shape=[
            jax.ShapeDtypeStruct((M, cout), jnp.float32),
            jax.ShapeDtypeStruct((8, cout), jnp.float32),
        ],
    )(zin, st, g, be, wT, brow, cq)


def _final_body(z_ref, st_ref, g_ref, be_ref, o_ref):
    y = _bn_apply(z_ref[...], st_ref, g_ref, be_ref)
    o_ref[...] = jnp.max(y.reshape(QB, K, 256), axis=1)


def _final(z3, st3, g3, be3):
    return pl.pallas_call(
        _final_body,
        grid=(NB,),
        in_specs=[
            pl.BlockSpec((RB, 256), lambda i: (i, 0)),
            pl.BlockSpec((8, 256), lambda i: (0, 0)),
            pl.BlockSpec((1, 256), lambda i: (0, 0)),
            pl.BlockSpec((1, 256), lambda i: (0, 0)),
        ],
        out_specs=pl.BlockSpec((QB, 256), lambda i: (i, 0)),
        out_shape=jax.ShapeDtypeStruct((B * S, 256), jnp.float32),
    )(z3, st3, g3, be3)


def kernel(xyz, points, W1, b1, g1, be1, W2, b2, g2, be2, W3, b3, g3, be3):
    xyzT = jnp.transpose(xyz, (2, 0, 1))
    far0 = jax.random.randint(jax.random.key(1), (B,), 0, N).astype(
        jnp.int32).reshape(B, 1)
    fps_idx, nx, ny, nz = _fps(xyzT, far0)
    new_xyz = jnp.stack([nx, ny, nz], axis=-1)
    qT = jnp.transpose(jnp.stack([nx, ny, nz], axis=0), (0, 2, 1))
    gidx = _knn(xyzT, qT)

    w1xT = jnp.transpose(W1[:, :3])
    w1pT = jnp.transpose(W1[:, 3:])
    t = _pretransform(xyz, points, w1xT, w1pT).reshape(B * N, 64)
    cq = _cq(new_xyz.reshape(B * S, 3), w1xT, b1.reshape(1, 64))

    tg = _gather_sc(t, gidx.reshape(M))

    st1 = _stats1(tg, cq)
    z2, st2 = _layer(tg, st1, g1.reshape(1, 64), be1.reshape(1, 64),
                     jnp.transpose(W2), b2.reshape(1, 128), cq, 64, 128, True)
    z3, st3 = _layer(z2, st2, g2.reshape(1, 128), be2.reshape(1, 128),
                     jnp.transpose(W3), b3.reshape(1, 256), cq, 128, 256,
                     False)
    new_points = _final(z3, st3, g3.reshape(1, 256),
                        be3.reshape(1, 256)).reshape(B, S, 256)
    return (new_xyz, new_points)


# speedup vs baseline: 8.6311x; 8.6311x over previous
import functools

import jax
import jax.numpy as jnp
from jax import lax
from jax.experimental import pallas as pl
from jax.experimental.pallas import tpu as pltpu
from jax.experimental.pallas import tpu_sc as plsc

B, N, S, K = 8, 4096, 512, 32
M = B * S * K
BIGI = 2**30


def _fps_body(xyzT_ref, far0_ref, nx_ref, ny_ref, nz_ref):
    x = xyzT_ref[0]
    y = xyzT_ref[1]
    z = xyzT_ref[2]
    iota = lax.broadcasted_iota(jnp.int32, (B, N), 1)

    def body(i, carry):
        far, dist = carry
        oh = iota == far
        cx = jnp.sum(jnp.where(oh, x, 0.0), axis=1, keepdims=True)
        cy = jnp.sum(jnp.where(oh, y, 0.0), axis=1, keepdims=True)
        cz = jnp.sum(jnp.where(oh, z, 0.0), axis=1, keepdims=True)
        nx_ref[pl.ds(i, 1)] = cx[None]
        ny_ref[pl.ds(i, 1)] = cy[None]
        nz_ref[pl.ds(i, 1)] = cz[None]
        dx = x - cx
        dy = y - cy
        dz = z - cz
        dn = dx * dx + dy * dy + dz * dz
        dist = jnp.minimum(dist, dn)
        m = jnp.max(dist, axis=1, keepdims=True)
        far = jnp.min(jnp.where(dist == m, iota, N), axis=1, keepdims=True)
        return far, dist

    far0 = far0_ref[...]
    dist0 = jnp.full((B, N), 1e10, dtype=jnp.float32)
    lax.fori_loop(0, S, body, (far0, dist0))


def _fps(xyzT, far0):
    return pl.pallas_call(
        _fps_body,
        out_shape=[
            jax.ShapeDtypeStruct((S, B, 1), jnp.float32),
            jax.ShapeDtypeStruct((S, B, 1), jnp.float32),
            jax.ShapeDtypeStruct((S, B, 1), jnp.float32),
        ],
    )(xyzT, far0)


def _knn_body(xyzB_ref, q_ref, idx_ref):
    b = pl.program_id(0)
    p = xyzB_ref[0]
    q = q_ref[0]
    a2 = jnp.sum(q * q, axis=1, keepdims=True)
    b2 = jnp.sum(p * p, axis=0, keepdims=True)
    qp = jnp.dot(q, p, preferred_element_type=jnp.float32)
    d = a2 + b2 - 2.0 * qp
    iota = lax.broadcasted_iota(jnp.int32, (S, N), 1)

    def body(r, dcur):
        m = jnp.min(dcur, axis=1, keepdims=True)
        am = jnp.min(jnp.where(dcur == m, iota, BIGI), axis=1, keepdims=True)
        idx_ref[pl.ds(r, 1)] = (am + b * N)[None, None]
        return jnp.where(iota == am, 3e38, dcur)

    lax.fori_loop(0, K, body, d)


def _knn(xyzB, q):
    return pl.pallas_call(
        _knn_body,
        grid=(B,),
        in_specs=[
            pl.BlockSpec((1, 3, N), lambda b: (b, 0, 0)),
            pl.BlockSpec((1, S, 3), lambda b: (b, 0, 0)),
        ],
        out_specs=pl.BlockSpec((K, 1, S, 1), lambda b: (0, b, 0, 0)),
        out_shape=jax.ShapeDtypeStruct((K, B, S, 1), jnp.int32),
    )(xyzB, q)


def _pret_body(xyz_ref, pts_ref, wx_ref, wp_ref, t_ref):
    t_ref[0] = (
        jnp.dot(xyz_ref[0], wx_ref[...], preferred_element_type=jnp.float32)
        + jnp.dot(pts_ref[0], wp_ref[...], preferred_element_type=jnp.float32)
    )


def _pretransform(xyz, points, w1xT, w1pT):
    return pl.pallas_call(
        _pret_body,
        grid=(B,),
        in_specs=[
            pl.BlockSpec((1, N, 3), lambda b: (b, 0, 0)),
            pl.BlockSpec((1, N, 64), lambda b: (b, 0, 0)),
            pl.BlockSpec((3, 64), lambda b: (0, 0)),
            pl.BlockSpec((64, 64), lambda b: (0, 0)),
        ],
        out_specs=pl.BlockSpec((1, N, 64), lambda b: (b, 0, 0)),
        out_shape=jax.ShapeDtypeStruct((B, N, 64), jnp.float32),
    )(xyz, points, w1xT, w1pT)


def _cq_body(ns_ref, wx_ref, b1_ref, cq_ref):
    cq_ref[...] = (
        jnp.dot(ns_ref[...], wx_ref[...], preferred_element_type=jnp.float32)
        - b1_ref[...]
    )


def _cq(ns, w1xT, b1row):
    return pl.pallas_call(
        _cq_body,
        out_shape=jax.ShapeDtypeStruct((B * S, 64), jnp.float32),
    )(ns, w1xT, b1row)


NWORK = 32
ROWS_W = M // NWORK
CH = 128
NCH = ROWS_W // CH


def _gather_sc(table, gidx):
    mesh = plsc.VectorSubcoreMesh(core_axis_name="c", subcore_axis_name="s")

    @functools.partial(
        pl.kernel,
        out_type=jax.ShapeDtypeStruct((M, 64), jnp.float32),
        mesh=mesh,
        compiler_params=pltpu.CompilerParams(use_tc_tiling_on_sc=False),
        scratch_types=[
            pltpu.VMEM((CH,), jnp.int32),
            pltpu.VMEM((CH, 64), jnp.float32),
            pltpu.SemaphoreType.DMA,
        ],
    )
    def k(table_hbm, idx_hbm, out_hbm, idx_v, rows_v, sem):
        wid = lax.axis_index("s") * 2 + lax.axis_index("c")
        base = wid * ROWS_W

        @pl.loop(0, NCH)
        def _(i):
            off = base + i * CH
            pltpu.sync_copy(idx_hbm.at[pl.ds(off, CH)], idx_v)
            pltpu.async_copy(table_hbm.at[idx_v], rows_v, sem).wait()
            pltpu.sync_copy(rows_v, out_hbm.at[pl.ds(off, CH)])

    return k(table, gidx)


RB = 4096
NB = M // RB
QB = RB // K


def _stats1_body(tg_ref, cq_ref, st_ref):
    @pl.when(pl.program_id(0) == 0)
    def _():
        st_ref[...] = jnp.zeros_like(st_ref)

    z = tg_ref[...].reshape(QB, K, 64) - cq_ref[...][:, None, :]
    s = jnp.sum(z, axis=(0, 1))[None, :]
    q = jnp.sum(z * z, axis=(0, 1))[None, :]
    st_ref[0:1, :] += s
    st_ref[1:2, :] += q


def _stats1(tg, cq):
    return pl.pallas_call(
        _stats1_body,
        grid=(NB,),
        in_specs=[
            pl.BlockSpec((RB, 64), lambda i: (i, 0)),
            pl.BlockSpec((QB, 64), lambda i: (i, 0)),
        ],
        out_specs=pl.BlockSpec((8, 64), lambda i: (0, 0)),
        out_shape=jax.ShapeDtypeStruct((8, 64), jnp.float32),
    )(tg, cq)


def _bn_apply(z, st_ref, g_ref, be_ref):
    s = st_ref[0:1, :]
    q = st_ref[1:2, :]
    mean = s / M
    var = q / M - mean * mean
    xh = (z - mean) / jnp.sqrt(var + 1e-5)
    return jnp.maximum(xh * g_ref[...] + be_ref[...], 0.0)


def _layer_body(in_ref, st_ref, g_ref, be_ref, w_ref, b_ref, cq_ref,
                z_ref, st2_ref, *, first):
    @pl.when(pl.program_id(0) == 0)
    def _():
        st2_ref[...] = jnp.zeros_like(st2_ref)

    zin = in_ref[...]
    if first:
        zin = (zin.reshape(QB, K, 64) - cq_ref[...][:, None, :]).reshape(
            RB, 64)
    y = _bn_apply(zin, st_ref, g_ref, be_ref)
    z = jnp.dot(y, w_ref[...], preferred_element_type=jnp.float32) + b_ref[...]
    z_ref[...] = z
    st2_ref[0:1, :] += jnp.sum(z, axis=0, keepdims=True)
    st2_ref[1:2, :] += jnp.sum(z * z, axis=0, keepdims=True)


def _layer(zin, st, g, be, wT, brow, cq, cin, cout, first):
    return pl.pallas_call(
        functools.partial(_layer_body, first=first),
        grid=(NB,),
        in_specs=[
            pl.BlockSpec((RB, cin), lambda i: (i, 0)),
            pl.BlockSpec((8, cin), lambda i: (0, 0)),
            pl.BlockSpec((1, cin), lambda i: (0, 0)),
            pl.BlockSpec((1, cin), lambda i: (0, 0)),
            pl.BlockSpec((cin, cout), lambda i: (0, 0)),
            pl.BlockSpec((1, cout), lambda i: (0, 0)),
            pl.BlockSpec((QB, 64), lambda i: (i, 0)),
        ],
        out_specs=[
            pl.BlockSpec((RB, cout), lambda i: (i, 0)),
            pl.BlockSpec((8, cout), lambda i: (0, 0)),
        ],
        out_shape=[
            jax.ShapeDtypeStruct((M, cout), jnp.float32),
            jax.ShapeDtypeStruct((8, cout), jnp.float32),
        ],
    )(zin, st, g, be, wT, brow, cq)


def _final_body(z_ref, st_ref, g_ref, be_ref, o_ref):
    y = _bn_apply(z_ref[...], st_ref, g_ref, be_ref)
    o_ref[...] = jnp.max(y.reshape(QB, K, 256), axis=1)


def _final(z3, st3, g3, be3):
    return pl.pallas_call(
        _final_body,
        grid=(NB,),
        in_specs=[
            pl.BlockSpec((RB, 256), lambda i: (i, 0)),
            pl.BlockSpec((8, 256), lambda i: (0, 0)),
            pl.BlockSpec((1, 256), lambda i: (0, 0)),
            pl.BlockSpec((1, 256), lambda i: (0, 0)),
        ],
        out_specs=pl.BlockSpec((QB, 256), lambda i: (i, 0)),
        out_shape=jax.ShapeDtypeStruct((B * S, 256), jnp.float32),
    )(z3, st3, g3, be3)


def kernel(xyz, points, W1, b1, g1, be1, W2, b2, g2, be2, W3, b3, g3, be3):
    xyzT = jnp.transpose(xyz, (2, 0, 1))
    far0 = jax.random.randint(jax.random.key(1), (B,), 0, N).astype(
        jnp.int32).reshape(B, 1)
    nx, ny, nz = _fps(xyzT, far0)
    new_xyz = jnp.transpose(
        jnp.concatenate([nx, ny, nz], axis=-1), (1, 0, 2))
    xyzB = jnp.transpose(xyz, (0, 2, 1))
    gidx = jnp.transpose(_knn(xyzB, new_xyz)[..., 0],
                         (1, 2, 0))

    w1xT = jnp.transpose(W1[:, :3])
    w1pT = jnp.transpose(W1[:, 3:])
    t = _pretransform(xyz, points, w1xT, w1pT).reshape(B * N, 64)
    cq = _cq(new_xyz.reshape(B * S, 3), w1xT, b1.reshape(1, 64))

    tg = _gather_sc(t, gidx.reshape(M))

    st1 = _stats1(tg, cq)
    z2, st2 = _layer(tg, st1, g1.reshape(1, 64), be1.reshape(1, 64),
                     jnp.transpose(W2), b2.reshape(1, 128), cq, 64, 128, True)
    z3, st3 = _layer(z2, st2, g2.reshape(1, 128), be2.reshape(1, 128),
                     jnp.transpose(W3), b3.reshape(1, 256), cq, 128, 256,
                     False)
    new_points = _final(z3, st3, g3.reshape(1, 256),
                        be3.reshape(1, 256)).reshape(B, S, 256)
    return (new_xyz, new_points)


# speedup vs baseline: 10.4117x; 1.2063x over previous
import functools

import jax
import jax.numpy as jnp
from jax import lax
from jax.experimental import pallas as pl
from jax.experimental.pallas import tpu as pltpu
from jax.experimental.pallas import tpu_sc as plsc

B, N, S, K = 8, 4096, 512, 32
M = B * S * K
BIGI = 2**30


def _fps_body(xyzT_ref, far0_ref, nx_ref, ny_ref, nz_ref):
    x = xyzT_ref[0]
    y = xyzT_ref[1]
    z = xyzT_ref[2]
    iota = lax.broadcasted_iota(jnp.int32, (B, N), 1)

    def body(i, carry):
        far, dist = carry
        oh = iota == far
        cx = jnp.sum(jnp.where(oh, x, 0.0), axis=1, keepdims=True)
        cy = jnp.sum(jnp.where(oh, y, 0.0), axis=1, keepdims=True)
        cz = jnp.sum(jnp.where(oh, z, 0.0), axis=1, keepdims=True)
        nx_ref[pl.ds(i, 1)] = cx[None]
        ny_ref[pl.ds(i, 1)] = cy[None]
        nz_ref[pl.ds(i, 1)] = cz[None]
        dx = x - cx
        dy = y - cy
        dz = z - cz
        dn = dx * dx + dy * dy + dz * dz
        dist = jnp.minimum(dist, dn)
        m = jnp.max(dist, axis=1, keepdims=True)
        far = jnp.min(jnp.where(dist == m, iota, N), axis=1, keepdims=True)
        return far, dist

    far0 = far0_ref[...]
    dist0 = jnp.full((B, N), 1e10, dtype=jnp.float32)
    lax.fori_loop(0, S, body, (far0, dist0))


def _fps(xyzT, far0):
    return pl.pallas_call(
        _fps_body,
        out_shape=[
            jax.ShapeDtypeStruct((S, B, 1), jnp.float32),
            jax.ShapeDtypeStruct((S, B, 1), jnp.float32),
            jax.ShapeDtypeStruct((S, B, 1), jnp.float32),
        ],
    )(xyzT, far0)


def _dists(xyzB_ref, q_ref):
    p = xyzB_ref[0]
    q = q_ref[0]
    a2 = jnp.sum(q * q, axis=1, keepdims=True)
    b2 = jnp.sum(p * p, axis=0, keepdims=True)
    qp = jnp.dot(q, p, preferred_element_type=jnp.float32)
    return a2 + b2 - 2.0 * qp


def _knn_brute_body(xyzB_ref, q_ref, idx_ref):
    b = pl.program_id(0)
    d = _dists(xyzB_ref, q_ref)
    iota = lax.broadcasted_iota(jnp.int32, (S, N), 1)

    def body(r, dcur):
        m = jnp.min(dcur, axis=1, keepdims=True)
        am = jnp.min(jnp.where(dcur == m, iota, BIGI), axis=1, keepdims=True)
        idx_ref[pl.ds(r, 1)] = (am + b * N)[None, None]
        return jnp.where(iota == am, 3e38, dcur)

    lax.fori_loop(0, K, body, d)


def _knn_brute(xyzB, q):
    return pl.pallas_call(
        _knn_brute_body,
        grid=(B,),
        in_specs=[
            pl.BlockSpec((1, 3, N), lambda b: (b, 0, 0)),
            pl.BlockSpec((1, S, 3), lambda b: (b, 0, 0)),
        ],
        out_specs=pl.BlockSpec((K, 1, S, 1), lambda b: (0, b, 0, 0)),
        out_shape=jax.ShapeDtypeStruct((K, B, S, 1), jnp.int32),
    )(xyzB, q)


NCHK = 32
KEEP = 8
SQ = 128


def _knn_fast_body(xyzB_ref, q_ref, idx_ref, flag_ref, d3_ref):
    b = pl.program_id(0)
    d = _dists(xyzB_ref, q_ref)
    d3_ref[...] = d.reshape(SQ, NCHK, 128)
    li = lax.broadcasted_iota(jnp.int32, (SQ, NCHK, 128), 2)
    ci = lax.broadcasted_iota(jnp.int32, (SQ, NCHK), 1)

    mvals, gidxs = [], []
    for _ in range(KEEP):
        d3 = d3_ref[...]
        m = jnp.min(d3, axis=2)
        am = jnp.min(jnp.where(d3 == m[:, :, None], li, BIGI), axis=2)
        d3_ref[...] = jnp.where(li == am[:, :, None], 3e38, d3)
        mvals.append(m)
        gidxs.append(ci * 128 + am)
    rv = jnp.concatenate(mvals, axis=1)
    rg = jnp.concatenate(gidxs, axis=1)

    cols = []
    tau = None
    for _ in range(K):
        m = jnp.min(rv, axis=1, keepdims=True)
        amg = jnp.min(jnp.where(rv == m, rg, BIGI), axis=1, keepdims=True)
        cols.append(amg + b * N)
        rv = jnp.where((rv == m) & (rg == amg), 3e38, rv)
        tau = m
    idx_ref[0] = jnp.concatenate(cols, axis=1)

    viol = jnp.where(mvals[KEEP - 1] < tau, 1, 0)
    fall = jnp.max(jnp.max(viol, axis=1, keepdims=True), axis=0,
                   keepdims=True)

    @pl.when((pl.program_id(0) == 0) & (pl.program_id(1) == 0))
    def _():
        flag_ref[...] = jnp.zeros_like(flag_ref)

    flag_ref[...] += fall


def _knn_fast(xyzB, q):
    return pl.pallas_call(
        _knn_fast_body,
        grid=(B, S // SQ),
        in_specs=[
            pl.BlockSpec((1, 3, N), lambda b, sb: (b, 0, 0)),
            pl.BlockSpec((1, SQ, 3), lambda b, sb: (b, sb, 0)),
        ],
        out_specs=[
            pl.BlockSpec((1, SQ, K), lambda b, sb: (b, sb, 0)),
            pl.BlockSpec((1, 1), lambda b, sb: (0, 0)),
        ],
        out_shape=[
            jax.ShapeDtypeStruct((B, S, K), jnp.int32),
            jax.ShapeDtypeStruct((1, 1), jnp.int32),
        ],
        scratch_shapes=[pltpu.VMEM((SQ, NCHK, 128), jnp.float32)],
    )(xyzB, q)


def _pret_body(xyz_ref, pts_ref, wx_ref, wp_ref, t_ref):
    t_ref[0] = (
        jnp.dot(xyz_ref[0], wx_ref[...], preferred_element_type=jnp.float32)
        + jnp.dot(pts_ref[0], wp_ref[...], preferred_element_type=jnp.float32)
    )


def _pretransform(xyz, points, w1xT, w1pT):
    return pl.pallas_call(
        _pret_body,
        grid=(B,),
        in_specs=[
            pl.BlockSpec((1, N, 3), lambda b: (b, 0, 0)),
            pl.BlockSpec((1, N, 64), lambda b: (b, 0, 0)),
            pl.BlockSpec((3, 64), lambda b: (0, 0)),
            pl.BlockSpec((64, 64), lambda b: (0, 0)),
        ],
        out_specs=pl.BlockSpec((1, N, 64), lambda b: (b, 0, 0)),
        out_shape=jax.ShapeDtypeStruct((B, N, 64), jnp.float32),
    )(xyz, points, w1xT, w1pT)


def _cq_body(ns_ref, wx_ref, b1_ref, cq_ref):
    cq_ref[...] = (
        jnp.dot(ns_ref[...], wx_ref[...], preferred_element_type=jnp.float32)
        - b1_ref[...]
    )


def _cq(ns, w1xT, b1row):
    return pl.pallas_call(
        _cq_body,
        out_shape=jax.ShapeDtypeStruct((B * S, 64), jnp.float32),
    )(ns, w1xT, b1row)


NWORK = 32
ROWS_W = M // NWORK
CH = 128
NCH = ROWS_W // CH


def _gather_sc(table, gidx):
    mesh = plsc.VectorSubcoreMesh(core_axis_name="c", subcore_axis_name="s")

    @functools.partial(
        pl.kernel,
        out_type=jax.ShapeDtypeStruct((M, 64), jnp.float32),
        mesh=mesh,
        compiler_params=pltpu.CompilerParams(use_tc_tiling_on_sc=False),
        scratch_types=[
            pltpu.VMEM((CH,), jnp.int32),
            pltpu.VMEM((CH, 64), jnp.float32),
            pltpu.SemaphoreType.DMA,
        ],
    )
    def k(table_hbm, idx_hbm, out_hbm, idx_v, rows_v, sem):
        wid = lax.axis_index("s") * 2 + lax.axis_index("c")
        base = wid * ROWS_W

        @pl.loop(0, NCH)
        def _(i):
            off = base + i * CH
            pltpu.sync_copy(idx_hbm.at[pl.ds(off, CH)], idx_v)
            pltpu.async_copy(table_hbm.at[idx_v], rows_v, sem).wait()
            pltpu.sync_copy(rows_v, out_hbm.at[pl.ds(off, CH)])

    return k(table, gidx)


RB = 4096
NB = M // RB
QB = RB // K


def _stats1_body(tg_ref, cq_ref, st_ref):
    @pl.when(pl.program_id(0) == 0)
    def _():
        st_ref[...] = jnp.zeros_like(st_ref)

    z = tg_ref[...].reshape(QB, K, 64) - cq_ref[...][:, None, :]
    s = jnp.sum(z, axis=(0, 1))[None, :]
    q = jnp.sum(z * z, axis=(0, 1))[None, :]
    st_ref[0:1, :] += s
    st_ref[1:2, :] += q


def _stats1(tg, cq):
    return pl.pallas_call(
        _stats1_body,
        grid=(NB,),
        in_specs=[
            pl.BlockSpec((RB, 64), lambda i: (i, 0)),
            pl.BlockSpec((QB, 64), lambda i: (i, 0)),
        ],
        out_specs=pl.BlockSpec((8, 64), lambda i: (0, 0)),
        out_shape=jax.ShapeDtypeStruct((8, 64), jnp.float32),
    )(tg, cq)


def _bn_apply(z, st_ref, g_ref, be_ref):
    s = st_ref[0:1, :]
    q = st_ref[1:2, :]
    mean = s / M
    var = q / M - mean * mean
    xh = (z - mean) / jnp.sqrt(var + 1e-5)
    return jnp.maximum(xh * g_ref[...] + be_ref[...], 0.0)


def _layer_body(in_ref, st_ref, g_ref, be_ref, w_ref, b_ref, cq_ref,
                z_ref, st2_ref, *, first):
    @pl.when(pl.program_id(0) == 0)
    def _():
        st2_ref[...] = jnp.zeros_like(st2_ref)

    zin = in_ref[...]
    if first:
        zin = (zin.reshape(QB, K, 64) - cq_ref[...][:, None, :]).reshape(
            RB, 64)
    y = _bn_apply(zin, st_ref, g_ref, be_ref)
    z = jnp.dot(y, w_ref[...], preferred_element_type=jnp.float32) + b_ref[...]
    z_ref[...] = z
    st2_ref[0:1, :] += jnp.sum(z, axis=0, keepdims=True)
    st2_ref[1:2, :] += jnp.sum(z * z, axis=0, keepdims=True)


def _layer(zin, st, g, be, wT, brow, cq, cin, cout, first):
    return pl.pallas_call(
        functools.partial(_layer_body, first=first),
        grid=(NB,),
        in_specs=[
            pl.BlockSpec((RB, cin), lambda i: (i, 0)),
            pl.BlockSpec((8, cin), lambda i: (0, 0)),
            pl.BlockSpec((1, cin), lambda i: (0, 0)),
            pl.BlockSpec((1, cin), lambda i: (0, 0)),
            pl.BlockSpec((cin, cout), lambda i: (0, 0)),
            pl.BlockSpec((1, cout), lambda i: (0, 0)),
            pl.BlockSpec((QB, 64), lambda i: (i, 0)),
        ],
        out_specs=[
            pl.BlockSpec((RB, cout), lambda i: (i, 0)),
            pl.BlockSpec((8, cout), lambda i: (0, 0)),
        ],
        out_shape=[
            jax.ShapeDtypeStruct((M, cout), jnp.float32),
            jax.ShapeDtypeStruct((8, cout), jnp.float32),
        ],
    )(zin, st, g, be, wT, brow, cq)


def _final_body(z_ref, st_ref, g_ref, be_ref, o_ref):
    y = _bn_apply(z_ref[...], st_ref, g_ref, be_ref)
    o_ref[...] = jnp.max(y.reshape(QB, K, 256), axis=1)


def _final(z3, st3, g3, be3):
    return pl.pallas_call(
        _final_body,
        grid=(NB,),
        in_specs=[
            pl.BlockSpec((RB, 256), lambda i: (i, 0)),
            pl.BlockSpec((8, 256), lambda i: (0, 0)),
            pl.BlockSpec((1, 256), lambda i: (0, 0)),
            pl.BlockSpec((1, 256), lambda i: (0, 0)),
        ],
        out_specs=pl.BlockSpec((QB, 256), lambda i: (i, 0)),
        out_shape=jax.ShapeDtypeStruct((B * S, 256), jnp.float32),
    )(z3, st3, g3, be3)


def kernel(xyz, points, W1, b1, g1, be1, W2, b2, g2, be2, W3, b3, g3, be3):
    xyzT = jnp.transpose(xyz, (2, 0, 1))
    far0 = jax.random.randint(jax.random.key(1), (B,), 0, N).astype(
        jnp.int32).reshape(B, 1)
    nx, ny, nz = _fps(xyzT, far0)
    new_xyz = jnp.transpose(
        jnp.concatenate([nx, ny, nz], axis=-1), (1, 0, 2))
    xyzB = jnp.transpose(xyz, (0, 2, 1))
    fast_idx, flag = _knn_fast(xyzB, new_xyz)
    gidx = lax.cond(
        flag[0, 0] > 0,
        lambda: jnp.transpose(_knn_brute(xyzB, new_xyz)[..., 0], (1, 2, 0)),
        lambda: fast_idx)

    w1xT = jnp.transpose(W1[:, :3])
    w1pT = jnp.transpose(W1[:, 3:])
    t = _pretransform(xyz, points, w1xT, w1pT).reshape(B * N, 64)
    cq = _cq(new_xyz.reshape(B * S, 3), w1xT, b1.reshape(1, 64))

    tg = _gather_sc(t, gidx.reshape(M))

    st1 = _stats1(tg, cq)
    z2, st2 = _layer(tg, st1, g1.reshape(1, 64), be1.reshape(1, 64),
                     jnp.transpose(W2), b2.reshape(1, 128), cq, 64, 128, True)
    z3, st3 = _layer(z2, st2, g2.reshape(1, 128), be2.reshape(1, 128),
                     jnp.transpose(W3), b3.reshape(1, 256), cq, 128, 256,
                     False)
    new_points = _final(z3, st3, g3.reshape(1, 256),
                        be3.reshape(1, 256)).reshape(B, S, 256)
    return (new_xyz, new_points)


# speedup vs baseline: 10.5628x; 1.0145x over previous
import functools

import jax
import jax.numpy as jnp
from jax import lax
from jax.experimental import pallas as pl
from jax.experimental.pallas import tpu as pltpu
from jax.experimental.pallas import tpu_sc as plsc

B, N, S, K = 8, 4096, 512, 32
M = B * S * K
BIGI = 2**30


def _fps_body(xyzT_ref, far0_ref, nx_ref, ny_ref, nz_ref):
    x = xyzT_ref[0]
    y = xyzT_ref[1]
    z = xyzT_ref[2]
    iota = lax.broadcasted_iota(jnp.int32, (B, N), 1)

    def body(i, carry):
        far, dist = carry
        oh = iota == far
        cx = jnp.sum(jnp.where(oh, x, 0.0), axis=1, keepdims=True)
        cy = jnp.sum(jnp.where(oh, y, 0.0), axis=1, keepdims=True)
        cz = jnp.sum(jnp.where(oh, z, 0.0), axis=1, keepdims=True)
        nx_ref[pl.ds(i, 1)] = cx[None]
        ny_ref[pl.ds(i, 1)] = cy[None]
        nz_ref[pl.ds(i, 1)] = cz[None]
        dx = x - cx
        dy = y - cy
        dz = z - cz
        dn = dx * dx + dy * dy + dz * dz
        dist = jnp.minimum(dist, dn)
        m = jnp.max(dist, axis=1, keepdims=True)
        far = jnp.min(jnp.where(dist == m, iota, N), axis=1, keepdims=True)
        return far, dist

    far0 = far0_ref[...]
    dist0 = jnp.full((B, N), 1e10, dtype=jnp.float32)
    lax.fori_loop(0, S, body, (far0, dist0))


def _fps(xyzT, far0):
    return pl.pallas_call(
        _fps_body,
        out_shape=[
            jax.ShapeDtypeStruct((S, B, 1), jnp.float32),
            jax.ShapeDtypeStruct((S, B, 1), jnp.float32),
            jax.ShapeDtypeStruct((S, B, 1), jnp.float32),
        ],
    )(xyzT, far0)


def _dists(xyzB_ref, q_ref):
    p = xyzB_ref[0]
    q = q_ref[0]
    a2 = jnp.sum(q * q, axis=1, keepdims=True)
    b2 = jnp.sum(p * p, axis=0, keepdims=True)
    qp = jnp.dot(q, p, preferred_element_type=jnp.float32)
    return a2 + b2 - 2.0 * qp


def _knn_brute_body(xyzB_ref, q_ref, idx_ref):
    b = pl.program_id(0)
    d = _dists(xyzB_ref, q_ref)
    iota = lax.broadcasted_iota(jnp.int32, (S, N), 1)

    def body(r, dcur):
        m = jnp.min(dcur, axis=1, keepdims=True)
        am = jnp.min(jnp.where(dcur == m, iota, BIGI), axis=1, keepdims=True)
        idx_ref[pl.ds(r, 1)] = (am + b * N)[None, None]
        return jnp.where(iota == am, 3e38, dcur)

    lax.fori_loop(0, K, body, d)


def _knn_brute(xyzB, q):
    return pl.pallas_call(
        _knn_brute_body,
        grid=(B,),
        in_specs=[
            pl.BlockSpec((1, 3, N), lambda b: (b, 0, 0)),
            pl.BlockSpec((1, S, 3), lambda b: (b, 0, 0)),
        ],
        out_specs=pl.BlockSpec((K, 1, S, 1), lambda b: (0, b, 0, 0)),
        out_shape=jax.ShapeDtypeStruct((K, B, S, 1), jnp.int32),
    )(xyzB, q)


NCHK = 32
KEEP = 8
SQ = 128


def _knn_fast_body(xyzB_ref, q_ref, idx_ref, flag_ref, k3_ref):
    b = pl.program_id(0)
    d = _dists(xyzB_ref, q_ref)
    li = lax.broadcasted_iota(jnp.int32, (SQ, NCHK, 128), 2)
    ci = lax.broadcasted_iota(jnp.int32, (SQ, NCHK), 1)
    bits = lax.bitcast_convert_type(d.reshape(SQ, NCHK, 128), jnp.int32)
    k3_ref[...] = (bits & -128) | li

    mvals, gidxs = [], []
    for r in range(KEEP):
        k3 = k3_ref[...]
        m = jnp.min(k3, axis=2)
        if r != KEEP - 1:
            k3_ref[...] = jnp.where(k3 == m[:, :, None], 0x7FFFFFFF, k3)
        mvals.append(m)
        gidxs.append(ci * 128 + (m & 127))
    rv = jnp.concatenate(mvals, axis=1)
    rg = jnp.concatenate(gidxs, axis=1)

    cols = []
    tau = None
    for _ in range(K):
        m = jnp.min(rv, axis=1, keepdims=True)
        amg = jnp.min(jnp.where(rv == m, rg, BIGI), axis=1, keepdims=True)
        cols.append(amg + b * N)
        rv = jnp.where((rv == m) & (rg == amg), 0x7FFFFFFF, rv)
        tau = m
    idx_ref[0] = jnp.concatenate(cols, axis=1)

    viol = jnp.where(mvals[KEEP - 1] < tau, 1, 0)
    fall = jnp.max(jnp.max(viol, axis=1, keepdims=True), axis=0,
                   keepdims=True)

    @pl.when((pl.program_id(0) == 0) & (pl.program_id(1) == 0))
    def _():
        flag_ref[...] = jnp.zeros_like(flag_ref)

    flag_ref[...] += fall


def _knn_fast(xyzB, q):
    return pl.pallas_call(
        _knn_fast_body,
        grid=(B, S // SQ),
        in_specs=[
            pl.BlockSpec((1, 3, N), lambda b, sb: (b, 0, 0)),
            pl.BlockSpec((1, SQ, 3), lambda b, sb: (b, sb, 0)),
        ],
        out_specs=[
            pl.BlockSpec((1, SQ, K), lambda b, sb: (b, sb, 0)),
            pl.BlockSpec((1, 1), lambda b, sb: (0, 0)),
        ],
        out_shape=[
            jax.ShapeDtypeStruct((B, S, K), jnp.int32),
            jax.ShapeDtypeStruct((1, 1), jnp.int32),
        ],
        scratch_shapes=[pltpu.VMEM((SQ, NCHK, 128), jnp.int32)],
    )(xyzB, q)


def _pret_body(xyz_ref, pts_ref, wx_ref, wp_ref, t_ref):
    t_ref[0] = (
        jnp.dot(xyz_ref[0], wx_ref[...], preferred_element_type=jnp.float32)
        + jnp.dot(pts_ref[0], wp_ref[...], preferred_element_type=jnp.float32)
    )


def _pretransform(xyz, points, w1xT, w1pT):
    return pl.pallas_call(
        _pret_body,
        grid=(B,),
        in_specs=[
            pl.BlockSpec((1, N, 3), lambda b: (b, 0, 0)),
            pl.BlockSpec((1, N, 64), lambda b: (b, 0, 0)),
            pl.BlockSpec((3, 64), lambda b: (0, 0)),
            pl.BlockSpec((64, 64), lambda b: (0, 0)),
        ],
        out_specs=pl.BlockSpec((1, N, 64), lambda b: (b, 0, 0)),
        out_shape=jax.ShapeDtypeStruct((B, N, 64), jnp.float32),
    )(xyz, points, w1xT, w1pT)


def _cq_body(ns_ref, wx_ref, b1_ref, cq_ref):
    cq_ref[...] = (
        jnp.dot(ns_ref[...], wx_ref[...], preferred_element_type=jnp.float32)
        - b1_ref[...]
    )


def _cq(ns, w1xT, b1row):
    return pl.pallas_call(
        _cq_body,
        out_shape=jax.ShapeDtypeStruct((B * S, 64), jnp.float32),
    )(ns, w1xT, b1row)


NWORK = 32
ROWS_W = M // NWORK
CH = 128
NCH = ROWS_W // CH


def _gather_sc(table, gidx):
    mesh = plsc.VectorSubcoreMesh(core_axis_name="c", subcore_axis_name="s")

    @functools.partial(
        pl.kernel,
        out_type=jax.ShapeDtypeStruct((M, 64), jnp.float32),
        mesh=mesh,
        compiler_params=pltpu.CompilerParams(use_tc_tiling_on_sc=False),
        scratch_types=[
            pltpu.VMEM((CH,), jnp.int32),
            pltpu.VMEM((CH, 64), jnp.float32),
            pltpu.SemaphoreType.DMA,
        ],
    )
    def k(table_hbm, idx_hbm, out_hbm, idx_v, rows_v, sem):
        wid = lax.axis_index("s") * 2 + lax.axis_index("c")
        base = wid * ROWS_W

        @pl.loop(0, NCH)
        def _(i):
            off = base + i * CH
            pltpu.sync_copy(idx_hbm.at[pl.ds(off, CH)], idx_v)
            pltpu.async_copy(table_hbm.at[idx_v], rows_v, sem).wait()
            pltpu.sync_copy(rows_v, out_hbm.at[pl.ds(off, CH)])

    return k(table, gidx)


RB = 4096
NB = M // RB
QB = RB // K


def _stats1_body(tg_ref, cq_ref, st_ref):
    @pl.when(pl.program_id(0) == 0)
    def _():
        st_ref[...] = jnp.zeros_like(st_ref)

    z = tg_ref[...].reshape(QB, K, 64) - cq_ref[...][:, None, :]
    s = jnp.sum(z, axis=(0, 1))[None, :]
    q = jnp.sum(z * z, axis=(0, 1))[None, :]
    st_ref[0:1, :] += s
    st_ref[1:2, :] += q


def _stats1(tg, cq):
    return pl.pallas_call(
        _stats1_body,
        grid=(NB,),
        in_specs=[
            pl.BlockSpec((RB, 64), lambda i: (i, 0)),
            pl.BlockSpec((QB, 64), lambda i: (i, 0)),
        ],
        out_specs=pl.BlockSpec((8, 64), lambda i: (0, 0)),
        out_shape=jax.ShapeDtypeStruct((8, 64), jnp.float32),
    )(tg, cq)


def _bn_apply(z, st_ref, g_ref, be_ref):
    s = st_ref[0:1, :]
    q = st_ref[1:2, :]
    mean = s / M
    var = q / M - mean * mean
    xh = (z - mean) / jnp.sqrt(var + 1e-5)
    return jnp.maximum(xh * g_ref[...] + be_ref[...], 0.0)


def _layer_body(in_ref, st_ref, g_ref, be_ref, w_ref, b_ref, cq_ref,
                z_ref, st2_ref, *, first):
    @pl.when(pl.program_id(0) == 0)
    def _():
        st2_ref[...] = jnp.zeros_like(st2_ref)

    zin = in_ref[...]
    if first:
        zin = (zin.reshape(QB, K, 64) - cq_ref[...][:, None, :]).reshape(
            RB, 64)
    y = _bn_apply(zin, st_ref, g_ref, be_ref)
    z = jnp.dot(y, w_ref[...], preferred_element_type=jnp.float32) + b_ref[...]
    z_ref[...] = z
    st2_ref[0:1, :] += jnp.sum(z, axis=0, keepdims=True)
    st2_ref[1:2, :] += jnp.sum(z * z, axis=0, keepdims=True)


def _layer(zin, st, g, be, wT, brow, cq, cin, cout, first):
    return pl.pallas_call(
        functools.partial(_layer_body, first=first),
        grid=(NB,),
        in_specs=[
            pl.BlockSpec((RB, cin), lambda i: (i, 0)),
            pl.BlockSpec((8, cin), lambda i: (0, 0)),
            pl.BlockSpec((1, cin), lambda i: (0, 0)),
            pl.BlockSpec((1, cin), lambda i: (0, 0)),
            pl.BlockSpec((cin, cout), lambda i: (0, 0)),
            pl.BlockSpec((1, cout), lambda i: (0, 0)),
            pl.BlockSpec((QB, 64), lambda i: (i, 0)),
        ],
        out_specs=[
            pl.BlockSpec((RB, cout), lambda i: (i, 0)),
            pl.BlockSpec((8, cout), lambda i: (0, 0)),
        ],
        out_shape=[
            jax.ShapeDtypeStruct((M, cout), jnp.float32),
            jax.ShapeDtypeStruct((8, cout), jnp.float32),
        ],
    )(zin, st, g, be, wT, brow, cq)


def _final_body(z_ref, st_ref, g_ref, be_ref, o_ref):
    y = _bn_apply(z_ref[...], st_ref, g_ref, be_ref)
    o_ref[...] = jnp.max(y.reshape(QB, K, 256), axis=1)


def _final(z3, st3, g3, be3):
    return pl.pallas_call(
        _final_body,
        grid=(NB,),
        in_specs=[
            pl.BlockSpec((RB, 256), lambda i: (i, 0)),
            pl.BlockSpec((8, 256), lambda i: (0, 0)),
            pl.BlockSpec((1, 256), lambda i: (0, 0)),
            pl.BlockSpec((1, 256), lambda i: (0, 0)),
        ],
        out_specs=pl.BlockSpec((QB, 256), lambda i: (i, 0)),
        out_shape=jax.ShapeDtypeStruct((B * S, 256), jnp.float32),
    )(z3, st3, g3, be3)


def kernel(xyz, points, W1, b1, g1, be1, W2, b2, g2, be2, W3, b3, g3, be3):
    xyzT = jnp.transpose(xyz, (2, 0, 1))
    far0 = jax.random.randint(jax.random.key(1), (B,), 0, N).astype(
        jnp.int32).reshape(B, 1)
    nx, ny, nz = _fps(xyzT, far0)
    new_xyz = jnp.transpose(
        jnp.concatenate([nx, ny, nz], axis=-1), (1, 0, 2))
    xyzB = jnp.transpose(xyz, (0, 2, 1))
    fast_idx, flag = _knn_fast(xyzB, new_xyz)
    gidx = lax.cond(
        flag[0, 0] > 0,
        lambda: jnp.transpose(_knn_brute(xyzB, new_xyz)[..., 0], (1, 2, 0)),
        lambda: fast_idx)

    w1xT = jnp.transpose(W1[:, :3])
    w1pT = jnp.transpose(W1[:, 3:])
    t = _pretransform(xyz, points, w1xT, w1pT).reshape(B * N, 64)
    cq = _cq(new_xyz.reshape(B * S, 3), w1xT, b1.reshape(1, 64))

    tg = _gather_sc(t, gidx.reshape(M))

    st1 = _stats1(tg, cq)
    z2, st2 = _layer(tg, st1, g1.reshape(1, 64), be1.reshape(1, 64),
                     jnp.transpose(W2), b2.reshape(1, 128), cq, 64, 128, True)
    z3, st3 = _layer(z2, st2, g2.reshape(1, 128), be2.reshape(1, 128),
                     jnp.transpose(W3), b3.reshape(1, 256), cq, 128, 256,
                     False)
    new_points = _final(z3, st3, g3.reshape(1, 256),
                        be3.reshape(1, 256)).reshape(B, S, 256)
    return (new_xyz, new_points)


# speedup vs baseline: 10.9631x; 1.0379x over previous
import functools

import jax
import jax.numpy as jnp
from jax import lax
from jax.experimental import pallas as pl
from jax.experimental.pallas import tpu as pltpu
from jax.experimental.pallas import tpu_sc as plsc

B, N, S, K = 8, 4096, 512, 32
M = B * S * K
BIGI = 2**30


def _fps_body(xyzT_ref, far0_ref, nx_ref, ny_ref, nz_ref):
    x = xyzT_ref[0]
    y = xyzT_ref[1]
    z = xyzT_ref[2]
    iota = lax.broadcasted_iota(jnp.int32, (B, N), 1)

    def body(i, carry):
        far, dist = carry
        oh = iota == far
        cx = jnp.sum(jnp.where(oh, x, 0.0), axis=1, keepdims=True)
        cy = jnp.sum(jnp.where(oh, y, 0.0), axis=1, keepdims=True)
        cz = jnp.sum(jnp.where(oh, z, 0.0), axis=1, keepdims=True)
        nx_ref[pl.ds(i, 1)] = cx[None]
        ny_ref[pl.ds(i, 1)] = cy[None]
        nz_ref[pl.ds(i, 1)] = cz[None]
        dx = x - cx
        dy = y - cy
        dz = z - cz
        dn = dx * dx + dy * dy + dz * dz
        dist = jnp.minimum(dist, dn)
        m = jnp.max(dist, axis=1, keepdims=True)
        far = jnp.min(jnp.where(dist == m, iota, N), axis=1, keepdims=True)
        return far, dist

    far0 = far0_ref[...]
    dist0 = jnp.full((B, N), 1e10, dtype=jnp.float32)
    lax.fori_loop(0, S, body, (far0, dist0))


def _fps(xyzT, far0):
    return pl.pallas_call(
        _fps_body,
        out_shape=[
            jax.ShapeDtypeStruct((S, B, 1), jnp.float32),
            jax.ShapeDtypeStruct((S, B, 1), jnp.float32),
            jax.ShapeDtypeStruct((S, B, 1), jnp.float32),
        ],
    )(xyzT, far0)


def _dists(xyzB_ref, q_ref):
    p = xyzB_ref[0]
    q = q_ref[0]
    a2 = jnp.sum(q * q, axis=1, keepdims=True)
    b2 = jnp.sum(p * p, axis=0, keepdims=True)
    qp = jnp.dot(q, p, preferred_element_type=jnp.float32)
    return a2 + b2 - 2.0 * qp


def _knn_brute_body(xyzB_ref, q_ref, idx_ref):
    b = pl.program_id(0)
    d = _dists(xyzB_ref, q_ref)
    iota = lax.broadcasted_iota(jnp.int32, (S, N), 1)

    def body(r, dcur):
        m = jnp.min(dcur, axis=1, keepdims=True)
        am = jnp.min(jnp.where(dcur == m, iota, BIGI), axis=1, keepdims=True)
        idx_ref[pl.ds(r, 1)] = (am + b * N)[None, None]
        return jnp.where(iota == am, 3e38, dcur)

    lax.fori_loop(0, K, body, d)


def _knn_brute(xyzB, q):
    return pl.pallas_call(
        _knn_brute_body,
        grid=(B,),
        in_specs=[
            pl.BlockSpec((1, 3, N), lambda b: (b, 0, 0)),
            pl.BlockSpec((1, S, 3), lambda b: (b, 0, 0)),
        ],
        out_specs=pl.BlockSpec((K, 1, S, 1), lambda b: (0, b, 0, 0)),
        out_shape=jax.ShapeDtypeStruct((K, B, S, 1), jnp.int32),
    )(xyzB, q)


NCHK = 32
KEEP = 8
SQ = 128


def _knn_fast_body(xyzB_ref, q_ref, idx_ref, flag_ref, k3_ref):
    b = pl.program_id(0)
    d = _dists(xyzB_ref, q_ref)
    li = lax.broadcasted_iota(jnp.int32, (SQ, NCHK, 128), 2)
    ci = lax.broadcasted_iota(jnp.int32, (SQ, NCHK), 1)
    bits = lax.bitcast_convert_type(d.reshape(SQ, NCHK, 128), jnp.int32)
    k3_ref[...] = (bits & -128) | li

    mvals, gidxs = [], []
    for r in range(KEEP // 2):
        k3 = k3_ref[...]
        m = jnp.min(k3, axis=2)
        m2 = jnp.min(jnp.where(k3 == m[:, :, None], 0x7FFFFFFF, k3), axis=2)
        if r != KEEP // 2 - 1:
            k3_ref[...] = jnp.where(
                (k3 == m[:, :, None]) | (k3 == m2[:, :, None]),
                0x7FFFFFFF, k3)
        mvals.extend([m, m2])
        gidxs.extend([ci * 128 + (m & 127), ci * 128 + (m2 & 127)])
    rv = jnp.concatenate(mvals, axis=1)
    rg = jnp.concatenate(gidxs, axis=1)

    cols = []
    tau = None
    for _ in range(K // 2):
        m = jnp.min(rv, axis=1, keepdims=True)
        e1 = rv == m
        amg = jnp.min(jnp.where(e1, rg, BIGI), axis=1, keepdims=True)
        m2 = jnp.min(jnp.where(e1, 0x7FFFFFFF, rv), axis=1, keepdims=True)
        e2 = rv == m2
        amg2 = jnp.min(jnp.where(e2, rg, BIGI), axis=1, keepdims=True)
        cols.append(amg + b * N)
        cols.append(amg2 + b * N)
        rv = jnp.where((e1 & (rg == amg)) | (e2 & (rg == amg2)),
                       0x7FFFFFFF, rv)
        tau = m2
    idx_ref[0] = jnp.concatenate(cols, axis=1)

    viol = jnp.where(mvals[KEEP - 1] < tau, 1, 0)
    fall = jnp.max(jnp.max(viol, axis=1, keepdims=True), axis=0,
                   keepdims=True)

    @pl.when((pl.program_id(0) == 0) & (pl.program_id(1) == 0))
    def _():
        flag_ref[...] = jnp.zeros_like(flag_ref)

    flag_ref[...] += fall


def _knn_fast(xyzB, q):
    return pl.pallas_call(
        _knn_fast_body,
        grid=(B, S // SQ),
        in_specs=[
            pl.BlockSpec((1, 3, N), lambda b, sb: (b, 0, 0)),
            pl.BlockSpec((1, SQ, 3), lambda b, sb: (b, sb, 0)),
        ],
        out_specs=[
            pl.BlockSpec((1, SQ, K), lambda b, sb: (b, sb, 0)),
            pl.BlockSpec((1, 1), lambda b, sb: (0, 0)),
        ],
        out_shape=[
            jax.ShapeDtypeStruct((B, S, K), jnp.int32),
            jax.ShapeDtypeStruct((1, 1), jnp.int32),
        ],
        scratch_shapes=[pltpu.VMEM((SQ, NCHK, 128), jnp.int32)],
    )(xyzB, q)


def _pret_body(xyz_ref, pts_ref, wx_ref, wp_ref, t_ref):
    t_ref[0] = (
        jnp.dot(xyz_ref[0], wx_ref[...], preferred_element_type=jnp.float32)
        + jnp.dot(pts_ref[0], wp_ref[...], preferred_element_type=jnp.float32)
    )


def _pretransform(xyz, points, w1xT, w1pT):
    return pl.pallas_call(
        _pret_body,
        grid=(B,),
        in_specs=[
            pl.BlockSpec((1, N, 3), lambda b: (b, 0, 0)),
            pl.BlockSpec((1, N, 64), lambda b: (b, 0, 0)),
            pl.BlockSpec((3, 64), lambda b: (0, 0)),
            pl.BlockSpec((64, 64), lambda b: (0, 0)),
        ],
        out_specs=pl.BlockSpec((1, N, 64), lambda b: (b, 0, 0)),
        out_shape=jax.ShapeDtypeStruct((B, N, 64), jnp.float32),
    )(xyz, points, w1xT, w1pT)


def _cq_body(ns_ref, wx_ref, b1_ref, cq_ref):
    cq_ref[...] = (
        jnp.dot(ns_ref[...], wx_ref[...], preferred_element_type=jnp.float32)
        - b1_ref[...]
    )


def _cq(ns, w1xT, b1row):
    return pl.pallas_call(
        _cq_body,
        out_shape=jax.ShapeDtypeStruct((B * S, 64), jnp.float32),
    )(ns, w1xT, b1row)


NWORK = 32
ROWS_W = M // NWORK
CH = 128
NCH = ROWS_W // CH


def _gather_sc(table, gidx):
    mesh = plsc.VectorSubcoreMesh(core_axis_name="c", subcore_axis_name="s")

    @functools.partial(
        pl.kernel,
        out_type=jax.ShapeDtypeStruct((M, 64), jnp.float32),
        mesh=mesh,
        compiler_params=pltpu.CompilerParams(use_tc_tiling_on_sc=False),
        scratch_types=[
            pltpu.VMEM((CH,), jnp.int32),
            pltpu.VMEM((CH, 64), jnp.float32),
            pltpu.SemaphoreType.DMA,
        ],
    )
    def k(table_hbm, idx_hbm, out_hbm, idx_v, rows_v, sem):
        wid = lax.axis_index("s") * 2 + lax.axis_index("c")
        base = wid * ROWS_W

        @pl.loop(0, NCH)
        def _(i):
            off = base + i * CH
            pltpu.sync_copy(idx_hbm.at[pl.ds(off, CH)], idx_v)
            pltpu.async_copy(table_hbm.at[idx_v], rows_v, sem).wait()
            pltpu.sync_copy(rows_v, out_hbm.at[pl.ds(off, CH)])

    return k(table, gidx)


RB = 4096
NB = M // RB
QB = RB // K


def _stats1_body(tg_ref, cq_ref, st_ref):
    @pl.when(pl.program_id(0) == 0)
    def _():
        st_ref[...] = jnp.zeros_like(st_ref)

    z = tg_ref[...].reshape(QB, K, 64) - cq_ref[...][:, None, :]
    s = jnp.sum(z, axis=(0, 1))[None, :]
    q = jnp.sum(z * z, axis=(0, 1))[None, :]
    st_ref[0:1, :] += s
    st_ref[1:2, :] += q


def _stats1(tg, cq):
    return pl.pallas_call(
        _stats1_body,
        grid=(NB,),
        in_specs=[
            pl.BlockSpec((RB, 64), lambda i: (i, 0)),
            pl.BlockSpec((QB, 64), lambda i: (i, 0)),
        ],
        out_specs=pl.BlockSpec((8, 64), lambda i: (0, 0)),
        out_shape=jax.ShapeDtypeStruct((8, 64), jnp.float32),
    )(tg, cq)


def _bn_apply(z, st_ref, g_ref, be_ref):
    s = st_ref[0:1, :]
    q = st_ref[1:2, :]
    mean = s / M
    var = q / M - mean * mean
    xh = (z - mean) / jnp.sqrt(var + 1e-5)
    return jnp.maximum(xh * g_ref[...] + be_ref[...], 0.0)


def _layer_body(in_ref, st_ref, g_ref, be_ref, w_ref, b_ref, cq_ref,
                z_ref, st2_ref, *, first):
    @pl.when(pl.program_id(0) == 0)
    def _():
        st2_ref[...] = jnp.zeros_like(st2_ref)

    zin = in_ref[...]
    if first:
        zin = (zin.reshape(QB, K, 64) - cq_ref[...][:, None, :]).reshape(
            RB, 64)
    y = _bn_apply(zin, st_ref, g_ref, be_ref)
    z = jnp.dot(y, w_ref[...], preferred_element_type=jnp.float32) + b_ref[...]
    z_ref[...] = z
    st2_ref[0:1, :] += jnp.sum(z, axis=0, keepdims=True)
    st2_ref[1:2, :] += jnp.sum(z * z, axis=0, keepdims=True)


def _layer(zin, st, g, be, wT, brow, cq, cin, cout, first):
    return pl.pallas_call(
        functools.partial(_layer_body, first=first),
        grid=(NB,),
        in_specs=[
            pl.BlockSpec((RB, cin), lambda i: (i, 0)),
            pl.BlockSpec((8, cin), lambda i: (0, 0)),
            pl.BlockSpec((1, cin), lambda i: (0, 0)),
            pl.BlockSpec((1, cin), lambda i: (0, 0)),
            pl.BlockSpec((cin, cout), lambda i: (0, 0)),
            pl.BlockSpec((1, cout), lambda i: (0, 0)),
            pl.BlockSpec((QB, 64), lambda i: (i, 0)),
        ],
        out_specs=[
            pl.BlockSpec((RB, cout), lambda i: (i, 0)),
            pl.BlockSpec((8, cout), lambda i: (0, 0)),
        ],
        out_shape=[
            jax.ShapeDtypeStruct((M, cout), jnp.float32),
            jax.ShapeDtypeStruct((8, cout), jnp.float32),
        ],
    )(zin, st, g, be, wT, brow, cq)


def _final_body(z_ref, st_ref, g_ref, be_ref, o_ref):
    y = _bn_apply(z_ref[...], st_ref, g_ref, be_ref)
    o_ref[...] = jnp.max(y.reshape(QB, K, 256), axis=1)


def _final(z3, st3, g3, be3):
    return pl.pallas_call(
        _final_body,
        grid=(NB,),
        in_specs=[
            pl.BlockSpec((RB, 256), lambda i: (i, 0)),
            pl.BlockSpec((8, 256), lambda i: (0, 0)),
            pl.BlockSpec((1, 256), lambda i: (0, 0)),
            pl.BlockSpec((1, 256), lambda i: (0, 0)),
        ],
        out_specs=pl.BlockSpec((QB, 256), lambda i: (i, 0)),
        out_shape=jax.ShapeDtypeStruct((B * S, 256), jnp.float32),
    )(z3, st3, g3, be3)


def kernel(xyz, points, W1, b1, g1, be1, W2, b2, g2, be2, W3, b3, g3, be3):
    xyzT = jnp.transpose(xyz, (2, 0, 1))
    far0 = jax.random.randint(jax.random.key(1), (B,), 0, N).astype(
        jnp.int32).reshape(B, 1)
    nx, ny, nz = _fps(xyzT, far0)
    new_xyz = jnp.transpose(
        jnp.concatenate([nx, ny, nz], axis=-1), (1, 0, 2))
    xyzB = jnp.transpose(xyz, (0, 2, 1))
    fast_idx, flag = _knn_fast(xyzB, new_xyz)
    gidx = lax.cond(
        flag[0, 0] > 0,
        lambda: jnp.transpose(_knn_brute(xyzB, new_xyz)[..., 0], (1, 2, 0)),
        lambda: fast_idx)

    w1xT = jnp.transpose(W1[:, :3])
    w1pT = jnp.transpose(W1[:, 3:])
    t = _pretransform(xyz, points, w1xT, w1pT).reshape(B * N, 64)
    cq = _cq(new_xyz.reshape(B * S, 3), w1xT, b1.reshape(1, 64))

    tg = _gather_sc(t, gidx.reshape(M))

    st1 = _stats1(tg, cq)
    z2, st2 = _layer(tg, st1, g1.reshape(1, 64), be1.reshape(1, 64),
                     jnp.transpose(W2), b2.reshape(1, 128), cq, 64, 128, True)
    z3, st3 = _layer(z2, st2, g2.reshape(1, 128), be2.reshape(1, 128),
                     jnp.transpose(W3), b3.reshape(1, 256), cq, 128, 256,
                     False)
    new_points = _final(z3, st3, g3.reshape(1, 256),
                        be3.reshape(1, 256)).reshape(B, S, 256)
    return (new_xyz, new_points)


# speedup vs baseline: 11.7331x; 1.0702x over previous
import functools

import jax
import jax.numpy as jnp
from jax import lax
from jax.experimental import pallas as pl
from jax.experimental.pallas import tpu as pltpu
from jax.experimental.pallas import tpu_sc as plsc

B, N, S, K = 8, 4096, 512, 32
M = B * S * K
BIGI = 2**30


def _fps_body(xyzT_ref, far0_ref, nx_ref, ny_ref, nz_ref):
    x = xyzT_ref[0]
    y = xyzT_ref[1]
    z = xyzT_ref[2]
    iota = lax.broadcasted_iota(jnp.int32, (B, N), 1)

    def body(i, carry):
        far, dist = carry
        oh = iota == far
        cx = jnp.sum(jnp.where(oh, x, 0.0), axis=1, keepdims=True)
        cy = jnp.sum(jnp.where(oh, y, 0.0), axis=1, keepdims=True)
        cz = jnp.sum(jnp.where(oh, z, 0.0), axis=1, keepdims=True)
        nx_ref[pl.ds(i, 1)] = cx[None]
        ny_ref[pl.ds(i, 1)] = cy[None]
        nz_ref[pl.ds(i, 1)] = cz[None]
        dx = x - cx
        dy = y - cy
        dz = z - cz
        dn = dx * dx + dy * dy + dz * dz
        dist = jnp.minimum(dist, dn)
        m = jnp.max(dist, axis=1, keepdims=True)
        far = jnp.min(jnp.where(dist == m, iota, N), axis=1, keepdims=True)
        return far, dist

    far0 = far0_ref[...]
    dist0 = jnp.full((B, N), 1e10, dtype=jnp.float32)
    lax.fori_loop(0, S, body, (far0, dist0))


def _fps(xyzT, far0):
    return pl.pallas_call(
        _fps_body,
        out_shape=[
            jax.ShapeDtypeStruct((S, B, 1), jnp.float32),
            jax.ShapeDtypeStruct((S, B, 1), jnp.float32),
            jax.ShapeDtypeStruct((S, B, 1), jnp.float32),
        ],
    )(xyzT, far0)


def _dists(xyzB_ref, q_ref):
    p = xyzB_ref[0]
    q = q_ref[0]
    a2 = jnp.sum(q * q, axis=1, keepdims=True)
    b2 = jnp.sum(p * p, axis=0, keepdims=True)
    qp = jnp.dot(q, p, preferred_element_type=jnp.float32)
    return a2 + b2 - 2.0 * qp


def _knn_brute_body(xyzB_ref, q_ref, idx_ref):
    b = pl.program_id(0)
    d = _dists(xyzB_ref, q_ref)
    iota = lax.broadcasted_iota(jnp.int32, (S, N), 1)

    def body(r, dcur):
        m = jnp.min(dcur, axis=1, keepdims=True)
        am = jnp.min(jnp.where(dcur == m, iota, BIGI), axis=1, keepdims=True)
        idx_ref[pl.ds(r, 1)] = (am + b * N)[None, None]
        return jnp.where(iota == am, 3e38, dcur)

    lax.fori_loop(0, K, body, d)


def _knn_brute(xyzB, q):
    return pl.pallas_call(
        _knn_brute_body,
        grid=(B,),
        in_specs=[
            pl.BlockSpec((1, 3, N), lambda b: (b, 0, 0)),
            pl.BlockSpec((1, S, 3), lambda b: (b, 0, 0)),
        ],
        out_specs=pl.BlockSpec((K, 1, S, 1), lambda b: (0, b, 0, 0)),
        out_shape=jax.ShapeDtypeStruct((K, B, S, 1), jnp.int32),
    )(xyzB, q)


NCHK = 32
KEEP = 8
SQ = 256


def _knn_fast_body(xyzB_ref, q_ref, idx_ref, flag_ref, k3_ref):
    b = pl.program_id(0)
    d = _dists(xyzB_ref, q_ref)
    li = lax.broadcasted_iota(jnp.int32, (SQ, NCHK, 128), 2)
    ci = lax.broadcasted_iota(jnp.int32, (SQ, NCHK), 1)
    bits = lax.bitcast_convert_type(d.reshape(SQ, NCHK, 128), jnp.int32)
    k3_ref[...] = (bits & -128) | li

    mvals, gidxs = [], []
    for r in range(KEEP // 2):
        k3 = k3_ref[...]
        m = jnp.min(k3, axis=2)
        m2 = jnp.min(jnp.where(k3 == m[:, :, None], 0x7FFFFFFF, k3), axis=2)
        if r != KEEP // 2 - 1:
            k3_ref[...] = jnp.where(
                (k3 == m[:, :, None]) | (k3 == m2[:, :, None]),
                0x7FFFFFFF, k3)
        mvals.extend([m, m2])
        gidxs.extend([ci * 128 + (m & 127), ci * 128 + (m2 & 127)])
    rv = jnp.concatenate(mvals, axis=1)
    rg = jnp.concatenate(gidxs, axis=1)

    cols = []
    tau = None
    for _ in range(K // 2):
        m = jnp.min(rv, axis=1, keepdims=True)
        e1 = rv == m
        amg = jnp.min(jnp.where(e1, rg, BIGI), axis=1, keepdims=True)
        m2 = jnp.min(jnp.where(e1, 0x7FFFFFFF, rv), axis=1, keepdims=True)
        e2 = rv == m2
        amg2 = jnp.min(jnp.where(e2, rg, BIGI), axis=1, keepdims=True)
        cols.append(amg + b * N)
        cols.append(amg2 + b * N)
        rv = jnp.where((e1 & (rg == amg)) | (e2 & (rg == amg2)),
                       0x7FFFFFFF, rv)
        tau = m2
    idx_ref[0] = jnp.concatenate(cols, axis=1)

    viol = jnp.where(mvals[KEEP - 1] < tau, 1, 0)
    fall = jnp.max(jnp.max(viol, axis=1, keepdims=True), axis=0,
                   keepdims=True)

    @pl.when((pl.program_id(0) == 0) & (pl.program_id(1) == 0))
    def _():
        flag_ref[...] = jnp.zeros_like(flag_ref)

    flag_ref[...] += fall


def _knn_fast(xyzB, q):
    return pl.pallas_call(
        _knn_fast_body,
        grid=(B, S // SQ),
        in_specs=[
            pl.BlockSpec((1, 3, N), lambda b, sb: (b, 0, 0)),
            pl.BlockSpec((1, SQ, 3), lambda b, sb: (b, sb, 0)),
        ],
        out_specs=[
            pl.BlockSpec((1, SQ, K), lambda b, sb: (b, sb, 0)),
            pl.BlockSpec((1, 1), lambda b, sb: (0, 0)),
        ],
        out_shape=[
            jax.ShapeDtypeStruct((B, S, K), jnp.int32),
            jax.ShapeDtypeStruct((1, 1), jnp.int32),
        ],
        scratch_shapes=[pltpu.VMEM((SQ, NCHK, 128), jnp.int32)],
    )(xyzB, q)


def _pret_body(xyz_ref, pts_ref, wx_ref, wp_ref, t_ref):
    t_ref[0] = (
        jnp.dot(xyz_ref[0], wx_ref[...], preferred_element_type=jnp.float32)
        + jnp.dot(pts_ref[0], wp_ref[...], preferred_element_type=jnp.float32)
    )


def _pretransform(xyz, points, w1xT, w1pT):
    return pl.pallas_call(
        _pret_body,
        grid=(B,),
        in_specs=[
            pl.BlockSpec((1, N, 3), lambda b: (b, 0, 0)),
            pl.BlockSpec((1, N, 64), lambda b: (b, 0, 0)),
            pl.BlockSpec((3, 64), lambda b: (0, 0)),
            pl.BlockSpec((64, 64), lambda b: (0, 0)),
        ],
        out_specs=pl.BlockSpec((1, N, 64), lambda b: (b, 0, 0)),
        out_shape=jax.ShapeDtypeStruct((B, N, 64), jnp.float32),
    )(xyz, points, w1xT, w1pT)


def _cq_body(ns_ref, wx_ref, b1_ref, cq_ref):
    cq_ref[...] = (
        jnp.dot(ns_ref[...], wx_ref[...], preferred_element_type=jnp.float32)
        - b1_ref[...]
    )


def _cq(ns, w1xT, b1row):
    return pl.pallas_call(
        _cq_body,
        out_shape=jax.ShapeDtypeStruct((B * S, 64), jnp.float32),
    )(ns, w1xT, b1row)


NWORK = 32
ROWS_W = M // NWORK
CH = 128
NCH = ROWS_W // CH


def _gather_sc(table, gidx):
    mesh = plsc.VectorSubcoreMesh(core_axis_name="c", subcore_axis_name="s")

    @functools.partial(
        pl.kernel,
        out_type=jax.ShapeDtypeStruct((M, 64), jnp.float32),
        mesh=mesh,
        compiler_params=pltpu.CompilerParams(use_tc_tiling_on_sc=False),
        scratch_types=[
            pltpu.VMEM((CH,), jnp.int32),
            pltpu.VMEM((CH, 64), jnp.float32),
            pltpu.SemaphoreType.DMA,
        ],
    )
    def k(table_hbm, idx_hbm, out_hbm, idx_v, rows_v, sem):
        wid = lax.axis_index("s") * 2 + lax.axis_index("c")
        base = wid * ROWS_W

        @pl.loop(0, NCH)
        def _(i):
            off = base + i * CH
            pltpu.sync_copy(idx_hbm.at[pl.ds(off, CH)], idx_v)
            pltpu.async_copy(table_hbm.at[idx_v], rows_v, sem).wait()
            pltpu.sync_copy(rows_v, out_hbm.at[pl.ds(off, CH)])

    return k(table, gidx)


RB = 4096
NB = M // RB
QB = RB // K


def _stats1_body(tg_ref, cq_ref, st_ref):
    @pl.when(pl.program_id(0) == 0)
    def _():
        st_ref[...] = jnp.zeros_like(st_ref)

    z = tg_ref[...].reshape(QB, K, 64) - cq_ref[...][:, None, :]
    s = jnp.sum(z, axis=(0, 1))[None, :]
    q = jnp.sum(z * z, axis=(0, 1))[None, :]
    st_ref[0:1, :] += s
    st_ref[1:2, :] += q


def _stats1(tg, cq):
    return pl.pallas_call(
        _stats1_body,
        grid=(NB,),
        in_specs=[
            pl.BlockSpec((RB, 64), lambda i: (i, 0)),
            pl.BlockSpec((QB, 64), lambda i: (i, 0)),
        ],
        out_specs=pl.BlockSpec((8, 64), lambda i: (0, 0)),
        out_shape=jax.ShapeDtypeStruct((8, 64), jnp.float32),
    )(tg, cq)


def _bn_apply(z, st_ref, g_ref, be_ref):
    s = st_ref[0:1, :]
    q = st_ref[1:2, :]
    mean = s / M
    var = q / M - mean * mean
    xh = (z - mean) / jnp.sqrt(var + 1e-5)
    return jnp.maximum(xh * g_ref[...] + be_ref[...], 0.0)


def _layer_body(in_ref, st_ref, g_ref, be_ref, w_ref, b_ref, cq_ref,
                z_ref, st2_ref, *, first):
    @pl.when(pl.program_id(0) == 0)
    def _():
        st2_ref[...] = jnp.zeros_like(st2_ref)

    zin = in_ref[...]
    if first:
        zin = (zin.reshape(QB, K, 64) - cq_ref[...][:, None, :]).reshape(
            RB, 64)
    y = _bn_apply(zin, st_ref, g_ref, be_ref)
    z = jnp.dot(y.astype(jnp.bfloat16), w_ref[...].astype(jnp.bfloat16),
                preferred_element_type=jnp.float32) + b_ref[...]
    z_ref[...] = z
    st2_ref[0:1, :] += jnp.sum(z, axis=0, keepdims=True)
    st2_ref[1:2, :] += jnp.sum(z * z, axis=0, keepdims=True)


def _layer(zin, st, g, be, wT, brow, cq, cin, cout, first):
    return pl.pallas_call(
        functools.partial(_layer_body, first=first),
        grid=(NB,),
        in_specs=[
            pl.BlockSpec((RB, cin), lambda i: (i, 0)),
            pl.BlockSpec((8, cin), lambda i: (0, 0)),
            pl.BlockSpec((1, cin), lambda i: (0, 0)),
            pl.BlockSpec((1, cin), lambda i: (0, 0)),
            pl.BlockSpec((cin, cout), lambda i: (0, 0)),
            pl.BlockSpec((1, cout), lambda i: (0, 0)),
            pl.BlockSpec((QB, 64), lambda i: (i, 0)),
        ],
        out_specs=[
            pl.BlockSpec((RB, cout), lambda i: (i, 0)),
            pl.BlockSpec((8, cout), lambda i: (0, 0)),
        ],
        out_shape=[
            jax.ShapeDtypeStruct((M, cout), jnp.float32),
            jax.ShapeDtypeStruct((8, cout), jnp.float32),
        ],
    )(zin, st, g, be, wT, brow, cq)


def _final_body(z_ref, st_ref, g_ref, be_ref, o_ref):
    y = _bn_apply(z_ref[...], st_ref, g_ref, be_ref)
    o_ref[...] = jnp.max(y.reshape(QB, K, 256), axis=1)


def _final(z3, st3, g3, be3):
    return pl.pallas_call(
        _final_body,
        grid=(NB,),
        in_specs=[
            pl.BlockSpec((RB, 256), lambda i: (i, 0)),
            pl.BlockSpec((8, 256), lambda i: (0, 0)),
            pl.BlockSpec((1, 256), lambda i: (0, 0)),
            pl.BlockSpec((1, 256), lambda i: (0, 0)),
        ],
        out_specs=pl.BlockSpec((QB, 256), lambda i: (i, 0)),
        out_shape=jax.ShapeDtypeStruct((B * S, 256), jnp.float32),
    )(z3, st3, g3, be3)


def kernel(xyz, points, W1, b1, g1, be1, W2, b2, g2, be2, W3, b3, g3, be3):
    xyzT = jnp.transpose(xyz, (2, 0, 1))
    far0 = jax.random.randint(jax.random.key(1), (B,), 0, N).astype(
        jnp.int32).reshape(B, 1)
    nx, ny, nz = _fps(xyzT, far0)
    new_xyz = jnp.transpose(
        jnp.concatenate([nx, ny, nz], axis=-1), (1, 0, 2))
    xyzB = jnp.transpose(xyz, (0, 2, 1))
    fast_idx, flag = _knn_fast(xyzB, new_xyz)
    gidx = lax.cond(
        flag[0, 0] > 0,
        lambda: jnp.transpose(_knn_brute(xyzB, new_xyz)[..., 0], (1, 2, 0)),
        lambda: fast_idx)

    w1xT = jnp.transpose(W1[:, :3])
    w1pT = jnp.transpose(W1[:, 3:])
    t = _pretransform(xyz, points, w1xT, w1pT).reshape(B * N, 64)
    cq = _cq(new_xyz.reshape(B * S, 3), w1xT, b1.reshape(1, 64))

    tg = _gather_sc(t, gidx.reshape(M))

    st1 = _stats1(tg, cq)
    z2, st2 = _layer(tg, st1, g1.reshape(1, 64), be1.reshape(1, 64),
                     jnp.transpose(W2), b2.reshape(1, 128), cq, 64, 128, True)
    z3, st3 = _layer(z2, st2, g2.reshape(1, 128), be2.reshape(1, 128),
                     jnp.transpose(W3), b3.reshape(1, 256), cq, 128, 256,
                     False)
    new_points = _final(z3, st3, g3.reshape(1, 256),
                        be3.reshape(1, 256)).reshape(B, S, 256)
    return (new_xyz, new_points)
